# baseline (speedup 1.0000x reference)
import functools

import jax
import jax.numpy as jnp
from jax import lax
from jax.experimental import pallas as pl
from jax.experimental.pallas import tpu as pltpu
from jax.experimental.pallas import tpu_sc as plsc

_N = 10000
_E = 320000
_D = 128
_C = 40
_CP = 64
_NPD = 10240
_NC = 2
_NS = 16
_NW = _NC * _NS
_K = 128
_NCH = _E // _K
_CPW = _NCH // _NW
_XTRA = _NCH - _CPW * _NW
_STRIP = _N // _NS
_RB = 2000

_MESH = dict(core_axis_name="c", subcore_axis_name="s",
             num_cores=_NC, num_subcores=_NS)


def _zero_fill_2d(ref, nrows, dim):
    zeros32 = jnp.zeros((32,), jnp.bfloat16)
    per_row = dim // 32

    def body(i, carry):
        ref[i // per_row, pl.ds((i % per_row) * 32, 32)] = zeros32
        return carry

    lax.fori_loop(0, nrows * per_row, body, 0)


def _zero_fill_1d(ref, n):
    zeros16 = jnp.zeros((16,), jnp.float32)

    def body(i, carry):
        ref[pl.ds(i * 16, 16)] = zeros16
        return carry

    lax.fori_loop(0, n // 16, body, 0)


def _zero_acc_strip(zsrc, acc, strip, dim):
    nz = _STRIP // _K
    for z in range(nz):
        pltpu.sync_copy(zsrc, acc.at[pl.ds(strip + z * _K, _K), :])
    rem = _STRIP % _K
    if rem:
        pltpu.sync_copy(zsrc.at[pl.ds(0, rem), :],
                        acc.at[pl.ds(strip + nz * _K, rem), :])


def _write_out_strip(acc, out_hbm, c, strip, stage, dim):
    nz = _STRIP // _K
    for z in range(nz):
        r0 = strip + z * _K
        pltpu.sync_copy(acc.at[pl.ds(r0, _K), :], stage)
        pltpu.sync_copy(stage, out_hbm.at[c, pl.ds(r0, _K), :])
    rem = _STRIP % _K
    if rem:
        r0 = strip + nz * _K
        pltpu.sync_copy(acc.at[pl.ds(r0, rem), :],
                        stage.at[pl.ds(0, rem), :])
        pltpu.sync_copy(stage.at[pl.ds(0, rem), :],
                        out_hbm.at[c, pl.ds(r0, rem), :])


def _make_edge_scatter_full(dim, nslot):
    assert _CPW % nslot == 0
    mesh = plsc.VectorSubcoreMesh(**_MESH)

    @functools.partial(
        pl.kernel,
        out_type=jax.ShapeDtypeStruct((_NC, _N, dim), jnp.bfloat16),
        mesh=mesh,
        scratch_types=[
            pltpu.VMEM((_CPW + 1, _K), jnp.int32),
            pltpu.VMEM((_CPW + 1, _K), jnp.int32),
            [pltpu.VMEM((_K, dim), jnp.bfloat16) for _ in range(nslot)],
            pltpu.VMEM_SHARED((_N, dim), jnp.bfloat16),
            [pltpu.SemaphoreType.DMA for _ in range(nslot)],
            [pltpu.SemaphoreType.DMA for _ in range(nslot)],
        ],
        compiler_params=pltpu.CompilerParams(use_tc_tiling_on_sc=False),
    )
    def scatter_kernel(p_hbm, edge_hbm, out_hbm,
                       row_v, col_v, bufs, acc, gsems, ssems):
        c = lax.axis_index("c")
        s = lax.axis_index("s")
        wid = c * _NS + s
        strip = s * _STRIP
        cbase = wid * _CPW

        pltpu.sync_copy(edge_hbm.at[0, pl.ds(cbase, _CPW), :],
                        row_v.at[pl.ds(0, _CPW), :])
        pltpu.sync_copy(edge_hbm.at[1, pl.ds(cbase, _CPW), :],
                        col_v.at[pl.ds(0, _CPW), :])
        _zero_fill_2d(bufs[0], _K, dim)
        _zero_acc_strip(bufs[0], acc, strip, dim)
        plsc.subcore_barrier()

        for b in range(nslot):
            pltpu.async_copy(p_hbm.at[row_v.at[b]], bufs[b], gsems[b])

        def body(r, carry):
            handles = []
            for b in range(nslot):
                j = r * nslot + b
                pltpu.make_async_copy(p_hbm.at[row_v.at[j]], bufs[b],
                                      gsems[b]).wait()
                handles.append(pltpu.async_copy(
                    bufs[b], acc.at[col_v.at[j]], ssems[b], add=True))
            for b in range(nslot):
                handles[b].wait()
                j2 = r * nslot + b + nslot

                @pl.when(j2 < _CPW)
                def _():
                    pltpu.async_copy(p_hbm.at[row_v.at[j2]], bufs[b],
                                     gsems[b])
            return carry

        lax.fori_loop(0, _CPW // nslot, body, 0)

        @pl.when(wid < _XTRA)
        def _():
            cid = _NW * _CPW + wid
            pltpu.sync_copy(edge_hbm.at[0, cid, :], row_v.at[_CPW])
            pltpu.sync_copy(edge_hbm.at[1, cid, :], col_v.at[_CPW])
            pltpu.async_copy(p_hbm.at[row_v.at[_CPW]], bufs[0],
                             gsems[0]).wait()
            pltpu.sync_copy(bufs[0], acc.at[col_v.at[_CPW]], add=True)

        plsc.subcore_barrier()
        _write_out_strip(acc, out_hbm, c, strip, bufs[0], dim)

    return scatter_kernel


def _make_degree():
    mesh = plsc.VectorSubcoreMesh(**_MESH)
    fire = 6

    @functools.partial(
        pl.kernel,
        out_type=jax.ShapeDtypeStruct((_NC * _NPD,), jnp.float32),
        mesh=mesh,
        scratch_types=[
            pltpu.VMEM((_CPW + 1, _K), jnp.int32),
            pltpu.VMEM((_K,), jnp.float32),
            pltpu.VMEM((_NPD // _NS,), jnp.float32),
            pltpu.VMEM_SHARED((_NPD,), jnp.float32),
            pltpu.SemaphoreType.DMA,
        ],
        compiler_params=pltpu.CompilerParams(use_tc_tiling_on_sc=False),
    )
    def degree_kernel(edge_hbm, out_hbm, col_v, ones_v, stage_v, acc, sem):
        c = lax.axis_index("c")
        s = lax.axis_index("s")
        wid = c * _NS + s
        dstrip = _NPD // _NS
        strip = s * dstrip

        ones16 = jnp.ones((16,), jnp.float32)
        for i in range(_K // 16):
            ones_v[pl.ds(i * 16, 16)] = ones16
        pltpu.sync_copy(edge_hbm.at[1, pl.ds(wid * _CPW, _CPW), :],
                        col_v.at[pl.ds(0, _CPW), :])
        _zero_fill_1d(stage_v, dstrip)
        pltpu.sync_copy(stage_v, acc.at[pl.ds(strip, dstrip)])
        plsc.subcore_barrier()

        def body(r, carry):
            handles = [
                pltpu.async_copy(ones_v, acc.at[col_v.at[r * fire + b]],
                                 sem, add=True)
                for b in range(fire)
            ]
            for h in handles:
                h.wait()
            return carry

        lax.fori_loop(0, _CPW // fire, body, 0)

        @pl.when(wid < _XTRA)
        def _():
            cid = _NW * _CPW + wid
            pltpu.sync_copy(edge_hbm.at[1, cid, :], col_v.at[_CPW])
            pltpu.sync_copy(ones_v, acc.at[col_v.at[_CPW]], add=True)

        plsc.subcore_barrier()
        pltpu.sync_copy(acc.at[pl.ds(strip, dstrip)], stage_v)
        pltpu.sync_copy(stage_v, out_hbm.at[pl.ds(c * _NPD + strip, dstrip)])

    return degree_kernel


_edge_scatter_d = _make_edge_scatter_full(_D, 6)
_edge_scatter_c = _make_edge_scatter_full(_CP, 13)
_degree = _make_degree()


def _p1_body(x_ref, w1_ref, deg_ref, p1_ref, p1h_ref, dinv_ref):
    di = lax.rsqrt(deg_ref[...])
    p1 = di * jnp.dot(x_ref[...], w1_ref[...],
                      preferred_element_type=jnp.float32)
    p1_ref[...] = p1
    p1h_ref[...] = p1.astype(jnp.bfloat16)
    dinv_ref[...] = di


def _tc_p1(x, w1, deg_col):
    grid = _N // _RB
    return pl.pallas_call(
        _p1_body,
        grid=(grid,),
        in_specs=[
            pl.BlockSpec((_RB, _D), lambda i: (i, 0)),
            pl.BlockSpec((_D, _D), lambda i: (0, 0)),
            pl.BlockSpec((_RB, 1), lambda i: (i, 0)),
        ],
        out_specs=[
            pl.BlockSpec((_RB, _D), lambda i: (i, 0)),
            pl.BlockSpec((_RB, _D), lambda i: (i, 0)),
            pl.BlockSpec((_RB, 1), lambda i: (i, 0)),
        ],
        out_shape=[
            jax.ShapeDtypeStruct((_N, _D), jnp.float32),
            jax.ShapeDtypeStruct((_N, _D), jnp.bfloat16),
            jax.ShapeDtypeStruct((_N, 1), jnp.float32),
        ],
    )(x, w1, deg_col)


def _p2_body(s1_ref, p1_ref, dinv_ref, b1_ref, w2_ref, p2_ref, p2h_ref):
    di = dinv_ref[...]
    s1 = s1_ref[0] + s1_ref[1]
    h = di * (s1 + p1_ref[...]) + b1_ref[...]
    h = jnp.maximum(h, 0.0)
    p2 = di * jnp.dot(h, w2_ref[...], preferred_element_type=jnp.float32)
    p2_ref[...] = p2
    p2h_ref[...] = p2.astype(jnp.bfloat16)


def _tc_p2(s1, p1, dinv, b1_row, w2p):
    grid = _N // _RB
    return pl.pallas_call(
        _p2_body,
        grid=(grid,),
        in_specs=[
            pl.BlockSpec((2, _RB, _D), lambda i: (0, i, 0)),
            pl.BlockSpec((_RB, _D), lambda i: (i, 0)),
            pl.BlockSpec((_RB, 1), lambda i: (i, 0)),
            pl.BlockSpec((1, _D), lambda i: (0, 0)),
            pl.BlockSpec((_D, _CP), lambda i: (0, 0)),
        ],
        out_specs=[
            pl.BlockSpec((_RB, _CP), lambda i: (i, 0)),
            pl.BlockSpec((_RB, _CP), lambda i: (i, 0)),
        ],
        out_shape=[
            jax.ShapeDtypeStruct((_N, _CP), jnp.float32),
            jax.ShapeDtypeStruct((_N, _CP), jnp.bfloat16),
        ],
    )(s1, p1, dinv, b1_row, w2p)


def _final_body(s2_ref, p2_ref, dinv_ref, b2_ref, logp_ref, logits_ref):
    di = dinv_ref[...]
    s2 = s2_ref[0] + s2_ref[1]
    lg = di * (s2 + p2_ref[...]) + b2_ref[...]
    icol = lax.broadcasted_iota(jnp.int32, (_RB, _CP), 1)
    neg = jnp.float32(-jnp.inf)
    lm = jnp.where(icol < _C, lg, neg)
    m = jnp.max(lm, axis=1, keepdims=True)
    e = jnp.where(icol < _C, jnp.exp(lm - m), 0.0)
    lse = m + jnp.log(jnp.sum(e, axis=1, keepdims=True))
    logp_ref[...] = (lg - lse)[:, :_C]
    logits_ref[...] = lg[:, :_C]


def _tc_final(s2, p2, dinv, b2_row):
    grid = _N // _RB
    return pl.pallas_call(
        _final_body,
        grid=(grid,),
        in_specs=[
            pl.BlockSpec((2, _RB, _CP), lambda i: (0, i, 0)),
            pl.BlockSpec((_RB, _CP), lambda i: (i, 0)),
            pl.BlockSpec((_RB, 1), lambda i: (i, 0)),
            pl.BlockSpec((1, _CP), lambda i: (0, 0)),
        ],
        out_specs=[
            pl.BlockSpec((_RB, _C), lambda i: (i, 0)),
            pl.BlockSpec((_RB, _C), lambda i: (i, 0)),
        ],
        out_shape=[
            jax.ShapeDtypeStruct((_N, _C), jnp.float32),
            jax.ShapeDtypeStruct((_N, _C), jnp.float32),
        ],
    )(s2, p2, dinv, b2_row)


def kernel(x, edge_index, W1, b1, W2, b2):
    edges = edge_index.astype(jnp.int32).reshape(2, _NCH, _K)

    degp = _degree(edges)
    deg_col = (degp[:_N] + degp[_NPD:_NPD + _N] + 1.0).reshape(_N, 1)

    p1, p1h, dinv = _tc_p1(x, W1, deg_col)

    s1 = _edge_scatter_d(p1h, edges).astype(jnp.float32)

    b1_row = b1.reshape(1, _D)
    w2p = jnp.pad(W2, ((0, 0), (0, _CP - _C)))
    p2, p2h = _tc_p2(s1, p1, dinv, b1_row, w2p)

    s2 = _edge_scatter_c(p2h, edges).astype(jnp.float32)

    b2_row = jnp.pad(b2, (0, _CP - _C)).reshape(1, _CP)
    logp, logits = _tc_final(s2, p2, dinv, b2_row)

    return (logp, logits)
